# baseline (speedup 1.0000x reference)
import functools

import jax
import jax.numpy as jnp
from jax import lax
from jax.experimental import pallas as pl
from jax.experimental.pallas import tpu as pltpu
from jax.experimental.pallas import tpu_sc as plsc

B = 16384
D = 32
L = 16

_INFO = None


def _info():
    global _INFO
    if _INFO is None:
        _INFO = plsc.get_sparse_core_info()
    return _INFO


def _pack(ut_t, ft_t, ct_t):
    info = _info()
    nw = info.num_cores * info.num_subcores
    mesh = plsc.VectorSubcoreMesh(core_axis_name="c", subcore_axis_name="s")

    NSLAB_U, NSLAB_F = 586, 8
    UNITS = NSLAB_U + NSLAB_F + 1

    @functools.partial(
        pl.kernel,
        mesh=mesh,
        compiler_params=pltpu.CompilerParams(use_tc_tiling_on_sc=True, needs_layout_passes=False),
        out_type=(
            jax.ShapeDtypeStruct((NSLAB_U * 128, 128), jnp.float32),
            jax.ShapeDtypeStruct((NSLAB_F * 128, 128), jnp.float32),
            jax.ShapeDtypeStruct((96, 128), jnp.float32),
        ),
        scratch_types=[
            pltpu.VMEM((32, 512), jnp.float32),
            pltpu.VMEM((128, 128), jnp.float32),
        ],
    )
    def k(utt, ftt, ctt, up, fp, cp, in_v, slab_v):
        wid = lax.axis_index("s") * info.num_cores + lax.axis_index("c")
        iota = lax.iota(jnp.int32, L)

        def transpose_rows(npack):
            def prow(p, _):
                for q in range(4):
                    jb = jnp.broadcast_to(p * 4 + q, (L,)).astype(jnp.int32)
                    for h in range(2):
                        v = plsc.load_gather(in_v, [iota + L * h, jb])
                        slab_v[p, pl.ds(32 * q + L * h, L)] = v
                return _

            lax.fori_loop(0, npack, prow, None)

        def do_unit(u, _):
            @pl.when(u < NSLAB_U)
            def _u():
                s = pl.multiple_of(u * 512, 128)
                pltpu.sync_copy(utt.at[:, pl.ds(s, 512)], in_v)
                transpose_rows(128)
                pltpu.sync_copy(slab_v, up.at[pl.ds(pl.multiple_of(u * 128, 8), 128)])

            @pl.when(jnp.logical_and(u >= NSLAB_U, u < NSLAB_U + NSLAB_F))
            def _f():
                s = pl.multiple_of((u - NSLAB_U) * 512, 128)
                pltpu.sync_copy(ftt.at[:, pl.ds(s, 512)], in_v)
                transpose_rows(128)
                pltpu.sync_copy(
                    slab_v,
                    fp.at[pl.ds(pl.multiple_of((u - NSLAB_U) * 128, 8), 128)],
                )

            @pl.when(u == NSLAB_U + NSLAB_F)
            def _c():
                s = pl.multiple_of((u - NSLAB_U - NSLAB_F) * 384, 128)
                pltpu.sync_copy(ctt.at[:, pl.ds(s, 384)], in_v.at[:, pl.ds(0, 384)])
                transpose_rows(96)
                pltpu.sync_copy(slab_v.at[pl.ds(0, 96)], cp)

            return _

        def unit_loop(jj, _):
            return do_unit(wid + nw * jj, _)

        lax.fori_loop(0, (UNITS + nw - 1) // nw, unit_loop, None)

    return k(ut_t, ft_t, ct_t)


def _gather(user, feed, city, up, fp, cp):
    info = _info()
    nw = info.num_cores * info.num_subcores
    bpw = B // nw
    CH = 128

    mesh = plsc.VectorSubcoreMesh(core_axis_name="c", subcore_axis_name="s")

    @functools.partial(
        pl.kernel,
        mesh=mesh,
        compiler_params=pltpu.CompilerParams(use_tc_tiling_on_sc=True, needs_layout_passes=False),
        out_type=jax.ShapeDtypeStruct((B, 3 * D), jnp.float32),
        scratch_types=[
            pltpu.VMEM((bpw,), jnp.int32),
            pltpu.VMEM((bpw,), jnp.int32),
            pltpu.VMEM((bpw,), jnp.int32),
            pltpu.VMEM((CH,), jnp.int32),
            pltpu.VMEM((CH,), jnp.int32),
            pltpu.VMEM((CH,), jnp.int32),
            pltpu.VMEM((CH, 128), jnp.float32),
            pltpu.VMEM((CH, 128), jnp.float32),
            pltpu.VMEM((CH, 128), jnp.float32),
            pltpu.VMEM((CH, 3 * D), jnp.float32),
            pltpu.SemaphoreType.DMA,
            pltpu.SemaphoreType.DMA,
            pltpu.SemaphoreType.DMA,
        ],
    )
    def k(user_hbm, feed_hbm, city_hbm, up_hbm, fp_hbm, cp_hbm, out_hbm,
          uidx_v, fidx_v, cidx_v, ug_v, fg_v, cg_v,
          upk_v, fpk_v, cpk_v, comb_v, su, sf, sc):
        wid = lax.axis_index("s") * info.num_cores + lax.axis_index("c")
        base = wid * bpw
        pltpu.sync_copy(user_hbm.at[pl.ds(base, bpw)], uidx_v)
        pltpu.sync_copy(feed_hbm.at[pl.ds(base, bpw)], fidx_v)
        pltpu.sync_copy(city_hbm.at[pl.ds(base, bpw)], cidx_v)

        def chunk(c, _):
            j0 = c * CH

            def widx(g, _2):
                ug_v[pl.ds(g * L, L)] = uidx_v[pl.ds(j0 + g * L, L)] >> 2
                fg_v[pl.ds(g * L, L)] = fidx_v[pl.ds(j0 + g * L, L)] >> 2
                cg_v[pl.ds(g * L, L)] = cidx_v[pl.ds(j0 + g * L, L)] >> 2
                return _2

            lax.fori_loop(0, CH // L, widx, None)

            cu = pltpu.async_copy(up_hbm.at[ug_v], upk_v, su)
            cf = pltpu.async_copy(fp_hbm.at[fg_v], fpk_v, sf)
            cc = pltpu.async_copy(cp_hbm.at[cg_v], cpk_v, sc)

            def extract(idx_ref, pk_v, colbase):
                def grp(g, _2):
                    qv = idx_ref[pl.ds(j0 + g * L, L)] & 3
                    for lane in range(L):
                        q = qv[lane]
                        j = g * L + lane
                        for h in range(2):
                            v = pk_v[j, pl.ds(q * D + h * L, L)]
                            comb_v[j, pl.ds(colbase + h * L, L)] = v
                    return _2

                lax.fori_loop(0, CH // L, grp, None)

            cu.wait()
            extract(uidx_v, upk_v, 0)
            cf.wait()
            extract(fidx_v, fpk_v, D)
            cc.wait()
            extract(cidx_v, cpk_v, 2 * D)

            pltpu.sync_copy(comb_v, out_hbm.at[pl.ds(base + j0, CH)])
            return _

        lax.fori_loop(0, bpw // CH, chunk, None)

    return k(user, feed, city, up, fp, cp)


def kernel(user, feed, city, user_table, feed_table, city_table):
    up, fp, cp = _pack(user_table.T, feed_table.T, city_table.T)
    return _gather(user.astype(jnp.int32), feed.astype(jnp.int32),
                   city.astype(jnp.int32), up, fp, cp)


# speedup vs baseline: 1.0002x; 1.0002x over previous
import functools

import jax
import jax.numpy as jnp
from jax import lax
from jax.experimental import pallas as pl
from jax.experimental.pallas import tpu as pltpu
from jax.experimental.pallas import tpu_sc as plsc

B = 16384
D = 32
L = 16

_INFO = None


def _info():
    global _INFO
    if _INFO is None:
        _INFO = plsc.get_sparse_core_info()
    return _INFO


def _pack(ut_t, ft_t, ct_t):
    info = _info()
    nw = info.num_cores * info.num_subcores
    mesh = plsc.VectorSubcoreMesh(core_axis_name="c", subcore_axis_name="s")

    NSLAB_U, NSLAB_F = 586, 8
    UNITS = NSLAB_U + NSLAB_F + 1

    @functools.partial(
        pl.kernel,
        mesh=mesh,
        compiler_params=pltpu.CompilerParams(use_tc_tiling_on_sc=True, needs_layout_passes=False, disable_bounds_checks=True),
        out_type=(
            jax.ShapeDtypeStruct((NSLAB_U * 128, 128), jnp.float32),
            jax.ShapeDtypeStruct((NSLAB_F * 128, 128), jnp.float32),
            jax.ShapeDtypeStruct((96, 128), jnp.float32),
        ),
        scratch_types=[
            pltpu.VMEM((32, 512), jnp.float32),
            pltpu.VMEM((128, 128), jnp.float32),
        ],
    )
    def k(utt, ftt, ctt, up, fp, cp, in_v, slab_v):
        wid = lax.axis_index("s") * info.num_cores + lax.axis_index("c")
        iota = lax.iota(jnp.int32, L)

        def transpose_rows(npack):
            def prow(p, _):
                for q in range(4):
                    jb = jnp.broadcast_to(p * 4 + q, (L,)).astype(jnp.int32)
                    for h in range(2):
                        v = plsc.load_gather(in_v, [iota + L * h, jb])
                        slab_v[p, pl.ds(32 * q + L * h, L)] = v
                return _

            lax.fori_loop(0, npack, prow, None)

        def do_unit(u, _):
            @pl.when(u < NSLAB_U)
            def _u():
                s = pl.multiple_of(u * 512, 128)
                pltpu.sync_copy(utt.at[:, pl.ds(s, 512)], in_v)
                transpose_rows(128)
                pltpu.sync_copy(slab_v, up.at[pl.ds(pl.multiple_of(u * 128, 8), 128)])

            @pl.when(jnp.logical_and(u >= NSLAB_U, u < NSLAB_U + NSLAB_F))
            def _f():
                s = pl.multiple_of((u - NSLAB_U) * 512, 128)
                pltpu.sync_copy(ftt.at[:, pl.ds(s, 512)], in_v)
                transpose_rows(128)
                pltpu.sync_copy(
                    slab_v,
                    fp.at[pl.ds(pl.multiple_of((u - NSLAB_U) * 128, 8), 128)],
                )

            @pl.when(u == NSLAB_U + NSLAB_F)
            def _c():
                s = pl.multiple_of((u - NSLAB_U - NSLAB_F) * 384, 128)
                pltpu.sync_copy(ctt.at[:, pl.ds(s, 384)], in_v.at[:, pl.ds(0, 384)])
                transpose_rows(96)
                pltpu.sync_copy(slab_v.at[pl.ds(0, 96)], cp)

            return _

        def unit_loop(jj, _):
            return do_unit(wid + nw * jj, _)

        lax.fori_loop(0, (UNITS + nw - 1) // nw, unit_loop, None)

    return k(ut_t, ft_t, ct_t)


def _gather(user, feed, city, up, fp, cp):
    info = _info()
    nw = info.num_cores * info.num_subcores
    bpw = B // nw
    CH = 128

    mesh = plsc.VectorSubcoreMesh(core_axis_name="c", subcore_axis_name="s")

    @functools.partial(
        pl.kernel,
        mesh=mesh,
        compiler_params=pltpu.CompilerParams(use_tc_tiling_on_sc=True, needs_layout_passes=False, disable_bounds_checks=True),
        out_type=jax.ShapeDtypeStruct((B, 3 * D), jnp.float32),
        scratch_types=[
            pltpu.VMEM((bpw,), jnp.int32),
            pltpu.VMEM((bpw,), jnp.int32),
            pltpu.VMEM((bpw,), jnp.int32),
            pltpu.VMEM((CH,), jnp.int32),
            pltpu.VMEM((CH,), jnp.int32),
            pltpu.VMEM((CH,), jnp.int32),
            pltpu.VMEM((CH, 128), jnp.float32),
            pltpu.VMEM((CH, 128), jnp.float32),
            pltpu.VMEM((CH, 128), jnp.float32),
            pltpu.VMEM((CH, 3 * D), jnp.float32),
            pltpu.SemaphoreType.DMA,
            pltpu.SemaphoreType.DMA,
            pltpu.SemaphoreType.DMA,
        ],
    )
    def k(user_hbm, feed_hbm, city_hbm, up_hbm, fp_hbm, cp_hbm, out_hbm,
          uidx_v, fidx_v, cidx_v, ug_v, fg_v, cg_v,
          upk_v, fpk_v, cpk_v, comb_v, su, sf, sc):
        wid = lax.axis_index("s") * info.num_cores + lax.axis_index("c")
        base = wid * bpw
        pltpu.sync_copy(user_hbm.at[pl.ds(base, bpw)], uidx_v)
        pltpu.sync_copy(feed_hbm.at[pl.ds(base, bpw)], fidx_v)
        pltpu.sync_copy(city_hbm.at[pl.ds(base, bpw)], cidx_v)

        def chunk(c, _):
            j0 = c * CH

            def widx(g, _2):
                ug_v[pl.ds(g * L, L)] = uidx_v[pl.ds(j0 + g * L, L)] >> 2
                fg_v[pl.ds(g * L, L)] = fidx_v[pl.ds(j0 + g * L, L)] >> 2
                cg_v[pl.ds(g * L, L)] = cidx_v[pl.ds(j0 + g * L, L)] >> 2
                return _2

            lax.fori_loop(0, CH // L, widx, None)

            cu = pltpu.async_copy(up_hbm.at[ug_v], upk_v, su)
            cf = pltpu.async_copy(fp_hbm.at[fg_v], fpk_v, sf)
            cc = pltpu.async_copy(cp_hbm.at[cg_v], cpk_v, sc)

            def extract(idx_ref, pk_v, colbase):
                def grp(g, _2):
                    qv = idx_ref[pl.ds(j0 + g * L, L)] & 3
                    for lane in range(L):
                        q = qv[lane]
                        j = g * L + lane
                        for h in range(2):
                            v = pk_v[j, pl.ds(q * D + h * L, L)]
                            comb_v[j, pl.ds(colbase + h * L, L)] = v
                    return _2

                lax.fori_loop(0, CH // L, grp, None)

            cu.wait()
            extract(uidx_v, upk_v, 0)
            cf.wait()
            extract(fidx_v, fpk_v, D)
            cc.wait()
            extract(cidx_v, cpk_v, 2 * D)

            pltpu.sync_copy(comb_v, out_hbm.at[pl.ds(base + j0, CH)])
            return _

        lax.fori_loop(0, bpw // CH, chunk, None)

    return k(user, feed, city, up, fp, cp)


def kernel(user, feed, city, user_table, feed_table, city_table):
    up, fp, cp = _pack(user_table.T, feed_table.T, city_table.T)
    return _gather(user.astype(jnp.int32), feed.astype(jnp.int32),
                   city.astype(jnp.int32), up, fp, cp)


# speedup vs baseline: 1.9187x; 1.9182x over previous
import functools

import jax
import jax.numpy as jnp
from jax import lax
from jax.experimental import pallas as pl
from jax.experimental.pallas import tpu as pltpu
from jax.experimental.pallas import tpu_sc as plsc

B = 16384
D = 32
L = 16


def _sc_embed(user, feed, city, user_table, feed_table, city_table):
    info = plsc.get_sparse_core_info()
    nw = info.num_cores * info.num_subcores
    bpw = B // nw

    mesh = plsc.VectorSubcoreMesh(core_axis_name="c", subcore_axis_name="s")

    @functools.partial(
        pl.kernel,
        mesh=mesh,
        compiler_params=pltpu.CompilerParams(use_tc_tiling_on_sc=False),
        out_type=jax.ShapeDtypeStruct((3 * B, D), jnp.float32),
        scratch_types=[
            pltpu.VMEM((bpw,), jnp.int32),
            pltpu.VMEM((bpw,), jnp.int32),
            pltpu.VMEM((bpw,), jnp.int32),
            pltpu.VMEM((bpw,), jnp.int32),
            pltpu.VMEM((bpw,), jnp.int32),
            pltpu.VMEM((bpw,), jnp.int32),
            pltpu.VMEM((bpw, D), jnp.float32),
            pltpu.VMEM((bpw, D), jnp.float32),
            pltpu.VMEM((bpw, D), jnp.float32),
            pltpu.SemaphoreType.DMA,
            pltpu.SemaphoreType.DMA,
            pltpu.SemaphoreType.DMA,
            pltpu.SemaphoreType.DMA,
            pltpu.SemaphoreType.DMA,
            pltpu.SemaphoreType.DMA,
        ],
    )
    def k(user_hbm, feed_hbm, city_hbm, ut_hbm, ft_hbm, ct_hbm, out_hbm,
          uidx_v, fidx_v, cidx_v, udst_v, fdst_v, cdst_v,
          urows_v, frows_v, crows_v, su, sf, sc, pu, pf, pc):
        wid = lax.axis_index("s") * info.num_cores + lax.axis_index("c")
        base = wid * bpw
        pltpu.sync_copy(user_hbm.at[pl.ds(base, bpw)], uidx_v)
        pltpu.sync_copy(feed_hbm.at[pl.ds(base, bpw)], fidx_v)
        pltpu.sync_copy(city_hbm.at[pl.ds(base, bpw)], cidx_v)
        cu = pltpu.async_copy(ut_hbm.at[uidx_v], urows_v, su)
        cf = pltpu.async_copy(ft_hbm.at[fidx_v], frows_v, sf)
        cc = pltpu.async_copy(ct_hbm.at[cidx_v], crows_v, sc)

        tri_iota = lax.iota(jnp.int32, L) * 3
        for i in range(bpw // L):
            d = tri_iota + (3 * base + 3 * L * i)
            udst_v[pl.ds(L * i, L)] = d
            fdst_v[pl.ds(L * i, L)] = d + 1
            cdst_v[pl.ds(L * i, L)] = d + 2

        cu.wait()
        wu = pltpu.async_copy(urows_v, out_hbm.at[udst_v], pu)
        cf.wait()
        wf = pltpu.async_copy(frows_v, out_hbm.at[fdst_v], pf)
        cc.wait()
        wc = pltpu.async_copy(crows_v, out_hbm.at[cdst_v], pc)
        wu.wait()
        wf.wait()
        wc.wait()

    out3 = k(user, feed, city, user_table, feed_table, city_table)
    return out3.reshape(B, 3 * D)


def kernel(user, feed, city, user_table, feed_table, city_table):
    return _sc_embed(user.astype(jnp.int32), feed.astype(jnp.int32),
                     city.astype(jnp.int32), user_table, feed_table, city_table)
